# baseline (speedup 1.0000x reference)
import functools

import jax
import jax.numpy as jnp
from jax import lax
from jax.experimental import pallas as pl
from jax.experimental.pallas import tpu as pltpu
from jax.experimental.pallas import tpu_sc as plsc

B = 16384
K = 64
NROWS = 1000000
NC = 2
NS = 16
NW = NC * NS
BPW = B // NW
CHUNK = 128
NCHUNK = BPW // CHUNK
GROUPS = CHUNK // 16
VPC = CHUNK // 16

_mesh = plsc.VectorSubcoreMesh(core_axis_name="c", subcore_axis_name="s")


@functools.partial(
    pl.kernel,
    out_type=jax.ShapeDtypeStruct((NW, NCHUNK, CHUNK), jnp.float32),
    mesh=_mesh,
    compiler_params=pltpu.CompilerParams(use_tc_tiling_on_sc=False),
    scratch_types=[
        pltpu.VMEM((NCHUNK, CHUNK), jnp.int32),
        pltpu.VMEM((NCHUNK, CHUNK), jnp.int32),
        pltpu.VMEM((K // 2, NCHUNK, CHUNK), jnp.int32),
        pltpu.VMEM((K // 2, NCHUNK, CHUNK), jnp.int32),
        pltpu.VMEM((K, NCHUNK, CHUNK), jnp.float32),
        pltpu.VMEM((K, NCHUNK, CHUNK), jnp.float32),
        pltpu.VMEM((NCHUNK, CHUNK), jnp.float32),
        pltpu.VMEM((NCHUNK, CHUNK), jnp.float32),
        pltpu.VMEM((NCHUNK, CHUNK), jnp.float32),
        pltpu.SemaphoreType.DMA,
    ],
)
def _mf_sc(user_hbm, item_hbm, ue_hbm, ie_hbm, ub_hbm, ib_hbm, out_hbm,
           raw_u, raw_i, idx_u, idx_i, val_u, val_i, bias_u, bias_i,
           out_v, sem):
    wid = lax.axis_index("s") * NC + lax.axis_index("c")

    pltpu.sync_copy(user_hbm.at[wid], raw_u)
    pltpu.sync_copy(item_hbm.at[wid], raw_i)

    for c in range(NCHUNK):
        pltpu.async_copy(ub_hbm.at[raw_u.at[c]], bias_u.at[c], sem)
        pltpu.async_copy(ib_hbm.at[raw_i.at[c]], bias_i.at[c], sem)

    def drain(n):
        def drain_body(i, _):
            pltpu.make_async_copy(
                ue_hbm.at[pl.ds(0, CHUNK)], out_v.at[0], sem).wait()
            return _
        lax.fori_loop(0, n, drain_body, 0)

    for wave in range(2):
        kbase = wave * (K // 2)

        def k_body(j, _, kbase=kbase):
            kvec = jnp.broadcast_to((kbase + j) * NROWS, (16,)).astype(jnp.int32)
            for c in range(NCHUNK):
                for v in range(VPC):
                    sl = pl.ds(v * 16, 16)
                    idx_u[j, c, sl] = raw_u[c, sl] + kvec
                    idx_i[j, c, sl] = raw_i[c, sl] + kvec
            for c in range(NCHUNK):
                pltpu.async_copy(
                    ue_hbm.at[idx_u.at[j, c]], val_u.at[kbase + j, c], sem)
                pltpu.async_copy(
                    ie_hbm.at[idx_i.at[j, c]], val_i.at[kbase + j, c], sem)
            return _

        lax.fori_loop(0, K // 2, k_body, 0)
        drain((K // 2) * 2 * NCHUNK + (2 * NCHUNK if wave == 0 else 0))

    for c in range(NCHUNK):
        def g_body(g, _, c=c):
            sl = pl.ds(g * 16, 16)
            acc = bias_u[c, sl] + bias_i[c, sl]
            for k in range(K):
                acc = acc + val_u[k, c, sl] * val_i[k, c, sl]
            out_v[c, sl] = acc
            return _
        lax.fori_loop(0, GROUPS, g_body, 0)

    pltpu.sync_copy(out_v, out_hbm.at[wid])


def kernel(user, item, user_embedding, item_embedding, user_bias, item_bias):
    user = user.astype(jnp.int32).reshape(NW, NCHUNK, CHUNK)
    item = item.astype(jnp.int32).reshape(NW, NCHUNK, CHUNK)
    ue_flat = user_embedding.T.reshape(-1)
    ie_flat = item_embedding.T.reshape(-1)
    ub = user_bias.reshape(-1)
    ib = item_bias.reshape(-1)
    out = _mf_sc(user, item, ue_flat, ie_flat, ub, ib)
    return out.reshape(B)


# speedup vs baseline: 1.0001x; 1.0001x over previous
import functools

import jax
import jax.numpy as jnp
from jax import lax
from jax.experimental import pallas as pl
from jax.experimental.pallas import tpu as pltpu
from jax.experimental.pallas import tpu_sc as plsc

B = 16384
K = 64
NROWS = 1000000
NC = 2
NS = 16
NW = NC * NS
BPW = B // NW
CHUNK = 128
NCHUNK = BPW // CHUNK
GROUPS = CHUNK // 16
VPC = CHUNK // 16

_mesh = plsc.VectorSubcoreMesh(core_axis_name="c", subcore_axis_name="s")


@functools.partial(
    pl.kernel,
    out_type=jax.ShapeDtypeStruct((NW, NCHUNK, CHUNK), jnp.float32),
    mesh=_mesh,
    compiler_params=pltpu.CompilerParams(use_tc_tiling_on_sc=False),
    scratch_types=[
        pltpu.VMEM((K, NCHUNK, CHUNK), jnp.int32),
        pltpu.VMEM((K, NCHUNK, CHUNK), jnp.float32),
        pltpu.VMEM((K, NCHUNK, CHUNK), jnp.float32),
        pltpu.VMEM((NCHUNK, CHUNK), jnp.float32),
        pltpu.VMEM((NCHUNK, CHUNK), jnp.float32),
        pltpu.VMEM((NCHUNK, CHUNK), jnp.float32),
        pltpu.SemaphoreType.DMA,
    ],
)
def _mf_sc(user_hbm, item_hbm, ue_hbm, ie_hbm, ub_hbm, ib_hbm, out_hbm,
           idx, val_u, val_i, bias_u, bias_i, out_v, sem):
    wid = lax.axis_index("s") * NC + lax.axis_index("c")

    def drain(n):
        def drain_body(i, _):
            pltpu.make_async_copy(
                ue_hbm.at[pl.ds(0, CHUNK)], out_v.at[0], sem).wait()
            return _
        lax.fori_loop(0, n, drain_body, 0)

    for raw_hbm, tbl_hbm, b_hbm, val, bias in (
            (user_hbm, ue_hbm, ub_hbm, val_u, bias_u),
            (item_hbm, ie_hbm, ib_hbm, val_i, bias_i)):
        pltpu.sync_copy(raw_hbm.at[wid], idx.at[0])

        def build_body(j, _):
            kvec = jnp.broadcast_to(j * NROWS, (16,)).astype(jnp.int32)
            for c in range(NCHUNK):
                for v in range(VPC):
                    sl = pl.ds(v * 16, 16)
                    idx[j, c, sl] = idx[0, c, sl] + kvec
            return _

        lax.fori_loop(1, K, build_body, 0)

        for c in range(NCHUNK):
            pltpu.async_copy(b_hbm.at[idx.at[0, c]], bias.at[c], sem)

        def fire_body(k, _, tbl_hbm=tbl_hbm, val=val):
            for c in range(NCHUNK):
                pltpu.async_copy(tbl_hbm.at[idx.at[k, c]], val.at[k, c], sem)
            return _

        lax.fori_loop(0, K, fire_body, 0)
        drain(K * NCHUNK + NCHUNK)

    for c in range(NCHUNK):
        def g_body(g, _, c=c):
            sl = pl.ds(g * 16, 16)
            acc = bias_u[c, sl] + bias_i[c, sl]
            for k in range(K):
                acc = acc + val_u[k, c, sl] * val_i[k, c, sl]
            out_v[c, sl] = acc
            return _
        lax.fori_loop(0, GROUPS, g_body, 0)

    pltpu.sync_copy(out_v, out_hbm.at[wid])


def kernel(user, item, user_embedding, item_embedding, user_bias, item_bias):
    user = user.astype(jnp.int32).reshape(NW, NCHUNK, CHUNK)
    item = item.astype(jnp.int32).reshape(NW, NCHUNK, CHUNK)
    ue_flat = user_embedding.T.reshape(-1)
    ie_flat = item_embedding.T.reshape(-1)
    ub = user_bias.reshape(-1)
    ib = item_bias.reshape(-1)
    out = _mf_sc(user, item, ue_flat, ie_flat, ub, ib)
    return out.reshape(B)
